# Initial kernel scaffold; baseline (speedup 1.0000x reference)
import jax
import jax.numpy as jnp
from jax.experimental import pallas as pl


def kernel(fnode, fmess, agraph, bgraph, mask, W_z_w, W_z_b, W_r_w, U_r_w, U_r_b, W_h_w, W_h_b, W_o_w, W_o_b):
    raise NotImplementedError("write your pallas kernel here")


# speedup vs baseline: 5.6928x; 5.6928x over previous
import functools

import jax
import jax.numpy as jnp
from jax import lax
from jax.experimental import pallas as pl
from jax.experimental.pallas import tpu as pltpu
from jax.experimental.pallas import tpu_sc as plsc

H = 128
NB = 6

E_TOTAL = 160000
NW = 32
BLK = 32
NBLOCKS = E_TOTAL // BLK
STEPS = -(-NBLOCKS // NW)

N_TOTAL = 10000
NPAD = 10240
CHUNK_N = NPAD // NW
BLK_N = 80
NBLK_N = CHUNK_N // BLK_N

BM = 2000
BN = 1000


def _tc_pre_body(x_ref, wzxT, whxT, wrT, urT, bz, bh, bu,
                 xwz_ref, xwh_ref, r1_ref, h_ref, hu_ref):
    x = x_ref[...]
    xwz = jnp.dot(x, wzxT[...], preferred_element_type=jnp.float32) + bz[...]
    xwh = jnp.dot(x, whxT[...], preferred_element_type=jnp.float32) + bh[...]
    r1 = jnp.dot(x, wrT[...], preferred_element_type=jnp.float32)
    z = jax.nn.sigmoid(xwz)
    pre = jnp.tanh(xwh)
    h = z * pre
    row = pl.program_id(0) * x.shape[0] + lax.broadcasted_iota(jnp.int32, h.shape, 0)
    h = jnp.where(row == 0, 0.0, h)
    hu = jnp.dot(h, urT[...], preferred_element_type=jnp.float32) + bu[...]
    xwz_ref[...] = xwz
    xwh_ref[...] = xwh
    r1_ref[...] = r1
    h_ref[...] = h
    hu_ref[...] = hu


def _tc_upd_body(sh_ref, sg_ref, xwz_ref, xwh_ref, wzhT, whhT, urT, bu,
                 h_ref, hu_ref=None):
    sh = sh_ref[...]
    sg = sg_ref[...]
    z = jax.nn.sigmoid(xwz_ref[...] +
                       jnp.dot(sh, wzhT[...], preferred_element_type=jnp.float32))
    pre = jnp.tanh(xwh_ref[...] +
                   jnp.dot(sg, whhT[...], preferred_element_type=jnp.float32))
    h = (1.0 - z) * sh + z * pre
    row = pl.program_id(0) * sh.shape[0] + lax.broadcasted_iota(jnp.int32, h.shape, 0)
    h = jnp.where(row == 0, 0.0, h)
    h_ref[...] = h
    if hu_ref is not None:
        hu_ref[...] = jnp.dot(h, urT[...], preferred_element_type=jnp.float32) + bu[...]


def _tc_out_body(fn_ref, nei_ref, wonT, wohT, bo, mask_ref, out_ref):
    acc = (jnp.dot(fn_ref[...], wonT[...], preferred_element_type=jnp.float32) +
           jnp.dot(nei_ref[...], wohT[...], preferred_element_type=jnp.float32) +
           bo[...])
    out_ref[...] = jnp.maximum(acc, 0.0) * mask_ref[...]


def _row_spec(bm):
    return pl.BlockSpec((bm, H), lambda i: (i, 0))


def _full_spec():
    return pl.BlockSpec((H, H), lambda i: (0, 0))


def _bias_spec():
    return pl.BlockSpec((1, H), lambda i: (0, 0))


_MESH = plsc.VectorSubcoreMesh(core_axis_name="c", subcore_axis_name="s")


def _sc_mp_body(h_hbm, hu_hbm, r1_hbm, bidx_hbm, sumh_hbm, sumg_hbm,
                idx0, idx1, h0, h1, u0, u1, r10, r11, sh_v, sg_v, sem0, sem1):
    wid = lax.axis_index("s") * 2 + lax.axis_index("c")
    nblk = 156 + (wid < (NBLOCKS % NW)).astype(jnp.int32)
    idx = (idx0, idx1)
    hb = (h0, h1)
    ub = (u0, u1)
    r1b = (r10, r11)
    sems = (sem0, sem1)

    def issue(j, b):
        base = (wid + j * NW) * BLK
        pltpu.sync_copy(bidx_hbm.at[pl.ds(base * NB, BLK * NB)], idx[b])
        pltpu.async_copy(h_hbm.at[idx[b]], hb[b], sems[b])
        pltpu.async_copy(hu_hbm.at[idx[b]], ub[b], sems[b])
        pltpu.async_copy(r1_hbm.at[pl.ds(base, BLK)], r1b[b], sems[b])

    def wait_bufs(b):
        pltpu.make_async_copy(h_hbm.at[idx[b]], hb[b], sems[b]).wait()
        pltpu.make_async_copy(hu_hbm.at[idx[b]], ub[b], sems[b]).wait()
        pltpu.make_async_copy(r1_hbm.at[pl.ds(0, BLK)], r1b[b], sems[b]).wait()

    issue(0, 0)

    def pair(i, carry):
        for b in range(2):
            j = i * 2 + b

            @pl.when(j < nblk)
            def _step(j=j, b=b):
                @pl.when(j + 1 < nblk)
                def _prefetch(j=j, b=b):
                    issue(j + 1, 1 - b)

                wait_bufs(b)

                @plsc.parallel_loop(0, BLK * (H // 16), unroll=8)
                def _chunk(i, b=b):
                    e = i // (H // 16)
                    c = (i % (H // 16)) * 16
                    rr = r1b[b][e, pl.ds(c, 16)]
                    hv = [hb[b][e * NB + k, pl.ds(c, 16)] for k in range(NB)]
                    uv = [ub[b][e * NB + k, pl.ds(c, 16)] for k in range(NB)]
                    rv = [1.0 / (1.0 + jnp.exp(-(rr + uv[k]))) for k in range(NB)]
                    gv = [rv[k] * hv[k] for k in range(NB)]
                    sh = ((hv[0] + hv[1]) + (hv[2] + hv[3])) + (hv[4] + hv[5])
                    sg = ((gv[0] + gv[1]) + (gv[2] + gv[3])) + (gv[4] + gv[5])
                    sh_v[e, pl.ds(c, 16)] = sh
                    sg_v[e, pl.ds(c, 16)] = sg

                base = (wid + j * NW) * BLK
                pltpu.sync_copy(sh_v, sumh_hbm.at[pl.ds(base, BLK)])
                pltpu.sync_copy(sg_v, sumg_hbm.at[pl.ds(base, BLK)])

        return carry

    lax.fori_loop(0, (STEPS + 1) // 2, pair, 0)


_sc_mp = functools.partial(
    pl.kernel,
    _sc_mp_body,
    out_type=[jax.ShapeDtypeStruct((E_TOTAL, H), jnp.float32),
              jax.ShapeDtypeStruct((E_TOTAL, H), jnp.float32)],
    mesh=_MESH,
    scratch_types=[
        pltpu.VMEM((BLK * NB,), jnp.int32),
        pltpu.VMEM((BLK * NB,), jnp.int32),
        pltpu.VMEM((BLK * NB, H), jnp.float32),
        pltpu.VMEM((BLK * NB, H), jnp.float32),
        pltpu.VMEM((BLK * NB, H), jnp.float32),
        pltpu.VMEM((BLK * NB, H), jnp.float32),
        pltpu.VMEM((BLK, H), jnp.float32),
        pltpu.VMEM((BLK, H), jnp.float32),
        pltpu.VMEM((BLK, H), jnp.float32),
        pltpu.VMEM((BLK, H), jnp.float32),
        pltpu.SemaphoreType.DMA,
        pltpu.SemaphoreType.DMA,
    ],
)()


def _sc_ag_body(h_hbm, aidx_hbm, nei_hbm, idx_v, h_v, acc_v, sem):
    wid = lax.axis_index("s") * 2 + lax.axis_index("c")

    def blk(i, carry):
        base = wid * CHUNK_N + i * BLK_N
        pltpu.sync_copy(aidx_hbm.at[pl.ds(base * NB, BLK_N * NB)], idx_v)
        pltpu.async_copy(h_hbm.at[idx_v], h_v, sem).wait()

        def node(e, c1):
            def col(j, c2):
                c = j * 16
                s = jnp.zeros((16,), jnp.float32)
                for k in range(NB):
                    s = s + h_v[e * NB + k, pl.ds(c, 16)]
                acc_v[e, pl.ds(c, 16)] = s
                return c2

            return lax.fori_loop(0, H // 16, col, c1)

        lax.fori_loop(0, BLK_N, node, 0)
        pltpu.sync_copy(acc_v, nei_hbm.at[pl.ds(base, BLK_N)])
        return carry

    lax.fori_loop(0, NBLK_N, blk, 0)


_sc_ag = functools.partial(
    pl.kernel,
    _sc_ag_body,
    out_type=jax.ShapeDtypeStruct((NPAD, H), jnp.float32),
    mesh=_MESH,
    scratch_types=[
        pltpu.VMEM((BLK_N * NB,), jnp.int32),
        pltpu.VMEM((BLK_N * NB, H), jnp.float32),
        pltpu.VMEM((BLK_N, H), jnp.float32),
        pltpu.SemaphoreType.DMA,
    ],
)()


@jax.jit
def kernel(fnode, fmess, agraph, bgraph, mask,
           W_z_w, W_z_b, W_r_w, U_r_w, U_r_b, W_h_w, W_h_b, W_o_w, W_o_b):
    E = fmess.shape[0]
    N = fnode.shape[0]
    assert E == E_TOTAL and N == N_TOTAL

    wzxT = W_z_w[:, :H].T
    wzhT = W_z_w[:, H:].T
    whxT = W_h_w[:, :H].T
    whhT = W_h_w[:, H:].T
    wrT = W_r_w.T
    urT = U_r_w.T
    wonT = W_o_w[:, :H].T
    wohT = W_o_w[:, H:].T
    bz = W_z_b.reshape(1, H)
    bh = W_h_b.reshape(1, H)
    bu = U_r_b.reshape(1, H)
    bo = W_o_b.reshape(1, H)

    bidx = bgraph.reshape(-1)
    aidx = jnp.pad(agraph, ((0, NPAD - N), (0, 0))).reshape(-1)

    eshape = jax.ShapeDtypeStruct((E, H), jnp.float32)

    xwz, xwh, r1, h, hu = pl.pallas_call(
        _tc_pre_body,
        grid=(E // BM,),
        in_specs=[_row_spec(BM), _full_spec(), _full_spec(), _full_spec(),
                  _full_spec(), _bias_spec(), _bias_spec(), _bias_spec()],
        out_specs=[_row_spec(BM)] * 5,
        out_shape=[eshape] * 5,
    )(fmess, wzxT, whxT, wrT, urT, bz, bh, bu)

    upd_mid = pl.pallas_call(
        _tc_upd_body,
        grid=(E // BM,),
        in_specs=[_row_spec(BM)] * 4 + [_full_spec(), _full_spec(),
                                        _full_spec(), _bias_spec()],
        out_specs=[_row_spec(BM)] * 2,
        out_shape=[eshape] * 2,
    )

    upd_last = pl.pallas_call(
        functools.partial(_tc_upd_body),
        grid=(E // BM,),
        in_specs=[_row_spec(BM)] * 4 + [_full_spec(), _full_spec(),
                                        _full_spec(), _bias_spec()],
        out_specs=_row_spec(BM),
        out_shape=eshape,
    )

    sum_h, sum_g = _sc_mp(h, hu, r1, bidx)
    h, hu = upd_mid(sum_h, sum_g, xwz, xwh, wzhT, whhT, urT, bu)

    sum_h, sum_g = _sc_mp(h, hu, r1, bidx)
    h = upd_last(sum_h, sum_g, xwz, xwh, wzhT, whhT, urT, bu)

    nei = _sc_ag(h, aidx)[:N]

    out = pl.pallas_call(
        _tc_out_body,
        grid=(N // BN,),
        in_specs=[_row_spec(BN), _row_spec(BN), _full_spec(), _full_spec(),
                  _bias_spec(), pl.BlockSpec((BN, 1), lambda i: (i, 0))],
        out_specs=_row_spec(BN),
        out_shape=jax.ShapeDtypeStruct((N, H), jnp.float32),
    )(fnode, nei, wonT, wohT, bo, mask)

    return (out, h)


# speedup vs baseline: 7.9188x; 1.3910x over previous
import functools

import jax
import jax.numpy as jnp
from jax import lax
from jax.experimental import pallas as pl
from jax.experimental.pallas import tpu as pltpu
from jax.experimental.pallas import tpu_sc as plsc

H = 128
NB = 6

E_TOTAL = 160000
NW = 32
BLK = 32
NBLOCKS = E_TOTAL // BLK
STEPS = -(-NBLOCKS // NW)

N_TOTAL = 10000
NPAD = 10240
CHUNK_N = NPAD // NW
BLK_N = 64
NBLK_N = CHUNK_N // BLK_N

BM = 4000
BN = 1000


def _tc_pre_body(x_ref, wzxT, whxT, wrT, urT, bz, bh, bu,
                 r1_ref, h_ref, hu_ref):
    x = x_ref[...]
    xwz = jnp.dot(x, wzxT[...], preferred_element_type=jnp.float32) + bz[...]
    xwh = jnp.dot(x, whxT[...], preferred_element_type=jnp.float32) + bh[...]
    r1 = jnp.dot(x, wrT[...], preferred_element_type=jnp.float32)
    z = jax.nn.sigmoid(xwz)
    pre = jnp.tanh(xwh)
    h = z * pre
    row = pl.program_id(0) * x.shape[0] + lax.broadcasted_iota(jnp.int32, h.shape, 0)
    h = jnp.where(row == 0, 0.0, h)
    hu = jnp.dot(h, urT[...], preferred_element_type=jnp.float32) + bu[...]
    r1_ref[...] = r1
    h_ref[...] = h
    hu_ref[...] = hu


def _tc_upd_body(sh_ref, sg_ref, x_ref, wzxT, whxT, bz, bh, wzhT, whhT,
                 urT, bu, h_ref, hu_ref=None):
    sh = sh_ref[...]
    sg = sg_ref[...]
    x = x_ref[...]
    z = jax.nn.sigmoid(jnp.dot(x, wzxT[...], preferred_element_type=jnp.float32)
                       + bz[...] +
                       jnp.dot(sh, wzhT[...], preferred_element_type=jnp.float32))
    pre = jnp.tanh(jnp.dot(x, whxT[...], preferred_element_type=jnp.float32)
                   + bh[...] +
                   jnp.dot(sg, whhT[...], preferred_element_type=jnp.float32))
    h = (1.0 - z) * sh + z * pre
    row = pl.program_id(0) * sh.shape[0] + lax.broadcasted_iota(jnp.int32, h.shape, 0)
    h = jnp.where(row == 0, 0.0, h)
    h_ref[...] = h
    if hu_ref is not None:
        hu_ref[...] = jnp.dot(h, urT[...], preferred_element_type=jnp.float32) + bu[...]


def _tc_out_body(fn_ref, nei_ref, wonT, wohT, bo, mask_ref, out_ref):
    acc = (jnp.dot(fn_ref[...], wonT[...], preferred_element_type=jnp.float32) +
           jnp.dot(nei_ref[...], wohT[...], preferred_element_type=jnp.float32) +
           bo[...])
    out_ref[...] = jnp.maximum(acc, 0.0) * mask_ref[...]


def _row_spec(bm):
    return pl.BlockSpec((bm, H), lambda i: (i, 0))


def _full_spec():
    return pl.BlockSpec((H, H), lambda i: (0, 0))


def _bias_spec():
    return pl.BlockSpec((1, H), lambda i: (0, 0))


_MESH = plsc.VectorSubcoreMesh(core_axis_name="c", subcore_axis_name="s")


def _sc_mp_body(h_hbm, hu_hbm, r1_hbm, bidx_hbm, sumh_hbm, sumg_hbm,
                ix0, ix1, ix2, ix3, h0, h1, u0, u1, r10, r11,
                sh0, sh1, sg0, sg1,
                isem0, isem1, isem2, isem3, sem0, sem1, osem0, osem1):
    wid = lax.axis_index("s") * 2 + lax.axis_index("c")
    nblk = 156 + (wid < (NBLOCKS % NW)).astype(jnp.int32)
    idx = (ix0, ix1, ix2, ix3)
    isems = (isem0, isem1, isem2, isem3)
    hb = (h0, h1)
    ub = (u0, u1)
    r1b = (r10, r11)
    shb = (sh0, sh1)
    sgb = (sg0, sg1)
    sems = (sem0, sem1)
    osems = (osem0, osem1)

    def idx_fire(j, s):
        base = (wid + j * NW) * BLK
        pltpu.async_copy(bidx_hbm.at[pl.ds(base * NB, BLK * NB)], idx[s],
                         isems[s])

    def idx_wait(s):
        pltpu.make_async_copy(bidx_hbm.at[pl.ds(0, BLK * NB)], idx[s],
                              isems[s]).wait()

    def gather_fire(j, b, s):
        base = (wid + j * NW) * BLK
        pltpu.async_copy(h_hbm.at[idx[s]], hb[b], sems[b])
        pltpu.async_copy(hu_hbm.at[idx[s]], ub[b], sems[b])
        pltpu.async_copy(r1_hbm.at[pl.ds(base, BLK)], r1b[b], sems[b])

    def gather_wait(b, s):
        pltpu.make_async_copy(h_hbm.at[idx[s]], hb[b], sems[b]).wait()
        pltpu.make_async_copy(hu_hbm.at[idx[s]], ub[b], sems[b]).wait()
        pltpu.make_async_copy(r1_hbm.at[pl.ds(0, BLK)], r1b[b], sems[b]).wait()

    def wait_out(b):
        pltpu.make_async_copy(shb[b], sumh_hbm.at[pl.ds(0, BLK)], osems[b]).wait()
        pltpu.make_async_copy(sgb[b], sumg_hbm.at[pl.ds(0, BLK)], osems[b]).wait()

    idx_fire(0, 0)
    idx_fire(1, 1)
    idx_wait(0)
    gather_fire(0, 0, 0)

    def quad(i, carry):
        for q in range(4):
            j = i * 4 + q
            b = q % 2
            s = q
            s1 = (q + 1) % 4
            s2 = (q + 2) % 4

            @pl.when(j < nblk)
            def _step(j=j, b=b, s=s, s1=s1, s2=s2):
                @pl.when(j + 2 < nblk)
                def _pf_idx(j=j, s2=s2):
                    idx_fire(j + 2, s2)

                @pl.when(j + 1 < nblk)
                def _pf_gather(j=j, b=b, s1=s1):
                    idx_wait(s1)
                    gather_fire(j + 1, 1 - b, s1)

                gather_wait(b, s)

                @pl.when(j >= 2)
                def _drain(b=b):
                    wait_out(b)

                @plsc.parallel_loop(0, BLK * (H // 16), unroll=4)
                def _chunk(t, b=b):
                    e = t // (H // 16)
                    c = (t % (H // 16)) * 16
                    rr = r1b[b][e, pl.ds(c, 16)]
                    hv = [hb[b][e * NB + k, pl.ds(c, 16)] for k in range(NB)]
                    uv = [ub[b][e * NB + k, pl.ds(c, 16)] for k in range(NB)]
                    rv = [1.0 / (1.0 + jnp.exp(rr + uv[k])) for k in range(NB)]
                    gv = [rv[k] * hv[k] for k in range(NB)]
                    sh = ((hv[0] + hv[1]) + (hv[2] + hv[3])) + (hv[4] + hv[5])
                    sg = ((gv[0] + gv[1]) + (gv[2] + gv[3])) + (gv[4] + gv[5])
                    shb[b][e, pl.ds(c, 16)] = sh
                    sgb[b][e, pl.ds(c, 16)] = sg

                base = (wid + j * NW) * BLK
                pltpu.async_copy(shb[b], sumh_hbm.at[pl.ds(base, BLK)], osems[b])
                pltpu.async_copy(sgb[b], sumg_hbm.at[pl.ds(base, BLK)], osems[b])

        return carry

    lax.fori_loop(0, (STEPS + 3) // 4, quad, 0)
    wait_out(0)
    wait_out(1)


_sc_mp = functools.partial(
    pl.kernel,
    _sc_mp_body,
    out_type=[jax.ShapeDtypeStruct((E_TOTAL, H), jnp.float32),
              jax.ShapeDtypeStruct((E_TOTAL, H), jnp.float32)],
    mesh=_MESH,
    scratch_types=[
        pltpu.VMEM((BLK * NB,), jnp.int32),
        pltpu.VMEM((BLK * NB,), jnp.int32),
        pltpu.VMEM((BLK * NB,), jnp.int32),
        pltpu.VMEM((BLK * NB,), jnp.int32),
        pltpu.VMEM((BLK * NB, H), jnp.float32),
        pltpu.VMEM((BLK * NB, H), jnp.float32),
        pltpu.VMEM((BLK * NB, H), jnp.float32),
        pltpu.VMEM((BLK * NB, H), jnp.float32),
        pltpu.VMEM((BLK, H), jnp.float32),
        pltpu.VMEM((BLK, H), jnp.float32),
        pltpu.VMEM((BLK, H), jnp.float32),
        pltpu.VMEM((BLK, H), jnp.float32),
        pltpu.VMEM((BLK, H), jnp.float32),
        pltpu.VMEM((BLK, H), jnp.float32),
        pltpu.SemaphoreType.DMA,
        pltpu.SemaphoreType.DMA,
        pltpu.SemaphoreType.DMA,
        pltpu.SemaphoreType.DMA,
        pltpu.SemaphoreType.DMA,
        pltpu.SemaphoreType.DMA,
        pltpu.SemaphoreType.DMA,
        pltpu.SemaphoreType.DMA,
    ],
)()


def _sc_ag_body(h_hbm, aidx_hbm, nei_hbm,
                idx0, idx1, h0, h1, acc_v, sem0, sem1):
    wid = lax.axis_index("s") * 2 + lax.axis_index("c")
    idx = (idx0, idx1)
    hbuf = (h0, h1)
    sems = (sem0, sem1)

    def issue(j, b):
        base = wid * CHUNK_N + j * BLK_N
        pltpu.sync_copy(aidx_hbm.at[pl.ds(base * NB, BLK_N * NB)], idx[b])
        pltpu.async_copy(h_hbm.at[idx[b]], hbuf[b], sems[b])

    issue(0, 0)
    for j in range(NBLK_N):
        b = j % 2
        if j + 1 < NBLK_N:
            issue(j + 1, 1 - b)
        pltpu.make_async_copy(h_hbm.at[idx[b]], hbuf[b], sems[b]).wait()

        @plsc.parallel_loop(0, BLK_N * (H // 16), unroll=8)
        def _chunk(t, b=b):
            e = t // (H // 16)
            c = (t % (H // 16)) * 16
            hv = [hbuf[b][e * NB + k, pl.ds(c, 16)] for k in range(NB)]
            s = ((hv[0] + hv[1]) + (hv[2] + hv[3])) + (hv[4] + hv[5])
            acc_v[e, pl.ds(c, 16)] = s

        base = wid * CHUNK_N + j * BLK_N
        pltpu.sync_copy(acc_v, nei_hbm.at[pl.ds(base, BLK_N)])


_sc_ag = functools.partial(
    pl.kernel,
    _sc_ag_body,
    out_type=jax.ShapeDtypeStruct((NPAD, H), jnp.float32),
    mesh=_MESH,
    scratch_types=[
        pltpu.VMEM((BLK_N * NB,), jnp.int32),
        pltpu.VMEM((BLK_N * NB,), jnp.int32),
        pltpu.VMEM((BLK_N * NB, H), jnp.float32),
        pltpu.VMEM((BLK_N * NB, H), jnp.float32),
        pltpu.VMEM((BLK_N, H), jnp.float32),
        pltpu.SemaphoreType.DMA,
        pltpu.SemaphoreType.DMA,
    ],
)()


@jax.jit
def kernel(fnode, fmess, agraph, bgraph, mask,
           W_z_w, W_z_b, W_r_w, U_r_w, U_r_b, W_h_w, W_h_b, W_o_w, W_o_b):
    E = fmess.shape[0]
    N = fnode.shape[0]
    assert E == E_TOTAL and N == N_TOTAL

    wzxT = W_z_w[:, :H].T
    wzhT = W_z_w[:, H:].T
    whxT = W_h_w[:, :H].T
    whhT = W_h_w[:, H:].T
    wrT = -W_r_w.T
    urT = -U_r_w.T
    wonT = W_o_w[:, :H].T
    wohT = W_o_w[:, H:].T
    bz = W_z_b.reshape(1, H)
    bh = W_h_b.reshape(1, H)
    bu = -U_r_b.reshape(1, H)
    bo = W_o_b.reshape(1, H)

    bidx = bgraph.reshape(-1)
    aidx = jnp.pad(agraph, ((0, NPAD - N), (0, 0))).reshape(-1)

    eshape = jax.ShapeDtypeStruct((E, H), jnp.float32)

    r1, h, hu = pl.pallas_call(
        _tc_pre_body,
        grid=(E // BM,),
        in_specs=[_row_spec(BM), _full_spec(), _full_spec(), _full_spec(),
                  _full_spec(), _bias_spec(), _bias_spec(), _bias_spec()],
        out_specs=[_row_spec(BM)] * 3,
        out_shape=[eshape] * 3,
    )(fmess, wzxT, whxT, wrT, urT, bz, bh, bu)

    upd_specs = ([_row_spec(BM)] * 3 +
                 [_full_spec(), _full_spec(), _bias_spec(), _bias_spec(),
                  _full_spec(), _full_spec(), _full_spec(), _bias_spec()])

    upd_mid = pl.pallas_call(
        _tc_upd_body,
        grid=(E // BM,),
        in_specs=upd_specs,
        out_specs=[_row_spec(BM)] * 2,
        out_shape=[eshape] * 2,
    )

    upd_last = pl.pallas_call(
        functools.partial(_tc_upd_body),
        grid=(E // BM,),
        in_specs=upd_specs,
        out_specs=_row_spec(BM),
        out_shape=eshape,
    )

    sum_h, sum_g = _sc_mp(h, hu, r1, bidx)
    h, hu = upd_mid(sum_h, sum_g, fmess, wzxT, whxT, bz, bh, wzhT, whhT, urT, bu)

    sum_h, sum_g = _sc_mp(h, hu, r1, bidx)
    h = upd_last(sum_h, sum_g, fmess, wzxT, whxT, bz, bh, wzhT, whhT, urT, bu)

    nei = _sc_ag(h, aidx)[:N]

    out = pl.pallas_call(
        _tc_out_body,
        grid=(N // BN,),
        in_specs=[_row_spec(BN), _row_spec(BN), _full_spec(), _full_spec(),
                  _bias_spec(), pl.BlockSpec((BN, 1), lambda i: (i, 0))],
        out_specs=_row_spec(BN),
        out_shape=jax.ShapeDtypeStruct((N, H), jnp.float32),
    )(fnode, nei, wonT, wohT, bo, mask)

    return (out, h)
